# baseline (speedup 1.0000x reference)
import jax
import jax.numpy as jnp
from jax.experimental import pallas as pl

_ROWS = 16384
_COLS = 1024
_BLK = 2048
_TPAD = 128


def _fused_kernel(t_ref, sa_ref, sb_ref, x0_ref, eps_ref, o_ref):
    t_blk = t_ref[0, 0, :]
    cols = jax.lax.broadcasted_iota(jnp.int32, (t_blk.shape[0], _TPAD), 1)
    onehot = t_blk[:, None] == cols
    a = jnp.sum(jnp.where(onehot, sa_ref[0, :][None, :], 0.0), axis=1)
    b = jnp.sum(jnp.where(onehot, sb_ref[0, :][None, :], 0.0), axis=1)
    o_ref[...] = a[:, None] * x0_ref[...] + b[:, None] * eps_ref[...]


def kernel(x0, eps, t, alpha_hat):
    t32 = t.astype(jnp.int32).reshape(_ROWS // _BLK, 1, _BLK)
    nb = alpha_hat.shape[0]
    sa = jnp.sqrt(alpha_hat)
    sb = jnp.sqrt(1.0 - alpha_hat)
    pad = _TPAD - nb
    sa = jnp.pad(sa, (0, pad)).reshape(1, _TPAD)
    sb = jnp.pad(sb, (0, pad)).reshape(1, _TPAD)
    grid = (_ROWS // _BLK,)
    return pl.pallas_call(
        _fused_kernel,
        grid=grid,
        in_specs=[
            pl.BlockSpec((1, 1, _BLK), lambda i: (i, 0, 0)),
            pl.BlockSpec((1, _TPAD), lambda i: (0, 0)),
            pl.BlockSpec((1, _TPAD), lambda i: (0, 0)),
            pl.BlockSpec((_BLK, _COLS), lambda i: (i, 0)),
            pl.BlockSpec((_BLK, _COLS), lambda i: (i, 0)),
        ],
        out_specs=pl.BlockSpec((_BLK, _COLS), lambda i: (i, 0)),
        out_shape=jax.ShapeDtypeStruct((_ROWS, _COLS), jnp.float32),
    )(t32, sa, sb, x0, eps)
